# baseline (speedup 1.0000x reference)
import functools

import jax
import jax.numpy as jnp
from jax import lax
from jax.experimental import pallas as pl
from jax.experimental.pallas import tpu as pltpu
from jax.experimental.pallas import tpu_sc as plsc

_NC = 2
_NS = 16
_NW = _NC * _NS
_CHUNK = 64
_SLOTS = 4


@functools.partial(jax.jit, static_argnums=(2, 3))
def _sc_embedding_gather(tokens_3d, table_i32, b, d):
    d2 = d // 2
    groups = d // 32
    b_per_w = b // _NW
    n_chunks = b_per_w // _CHUNK
    mesh = plsc.VectorSubcoreMesh(core_axis_name="c", subcore_axis_name="s")

    @functools.partial(
        pl.kernel,
        mesh=mesh,
        out_type=jax.ShapeDtypeStruct((b, d), jnp.float32),
        scratch_types=(
            [pltpu.VMEM((n_chunks, _CHUNK), jnp.int32)]
            + [pltpu.VMEM((_CHUNK, d2), jnp.int32) for _ in range(_SLOTS)]
            + [pltpu.VMEM((_CHUNK, d), jnp.float32) for _ in range(_SLOTS)]
            + [pltpu.SemaphoreType.DMA for _ in range(2 * _SLOTS)]
        ),
    )
    def k(tok_hbm, tab_hbm, out_hbm, idx_v, *rest):
        ibuf = rest[:_SLOTS]
        fbuf = rest[_SLOTS:2 * _SLOTS]
        gsem = rest[2 * _SLOTS:3 * _SLOTS]
        wsem = rest[3 * _SLOTS:]
        wid = lax.axis_index("s") * _NC + lax.axis_index("c")
        base = wid * b_per_w

        pltpu.sync_copy(tok_hbm.at[wid], idx_v)

        def gather_start(c, p):
            pltpu.make_async_copy(tab_hbm.at[idx_v.at[c]], ibuf[p], gsem[p]).start()

        def gather_wait(p):
            pltpu.make_async_copy(tab_hbm.at[idx_v.at[0]], ibuf[p], gsem[p]).wait()

        def convert(p):
            def row_body(r, carry):
                for g in range(groups):
                    v = ibuf[p][r, pl.ds(16 * g, 16)]
                    fbuf[p][r, pl.ds(32 * g, 16)] = lax.bitcast_convert_type(
                        v << 16, jnp.float32)
                    fbuf[p][r, pl.ds(32 * g + 16, 16)] = lax.bitcast_convert_type(
                        v & (-65536), jnp.float32)
                return carry

            lax.fori_loop(0, _CHUNK, row_body, 0)

        def write_start(c, p):
            pltpu.make_async_copy(
                fbuf[p], out_hbm.at[pl.ds(base + c * _CHUNK, _CHUNK)], wsem[p]
            ).start()

        def write_wait(p):
            pltpu.make_async_copy(
                fbuf[p], out_hbm.at[pl.ds(base, _CHUNK)], wsem[p]
            ).wait()

        for p in range(_SLOTS):
            gather_start(p, p)

        for p in range(_SLOTS):
            gather_wait(p)
            convert(p)
            gather_start(_SLOTS + p, p)
            write_start(p, p)

        def body(j, carry):
            c0 = _SLOTS * j
            for p in range(_SLOTS):
                gather_wait(p)
                write_wait(p)
                convert(p)
                gather_start(lax.min(c0 + _SLOTS + p, n_chunks - 1), p)
                write_start(c0 + p, p)
            return carry

        lax.fori_loop(1, n_chunks // _SLOTS, body, 0)
        for p in range(_SLOTS):
            gather_wait(p)
            write_wait(p)

    return k(tokens_3d, table_i32)


def kernel(integer_tokens, token_embedding):
    bsz, seq = integer_tokens.shape
    v, d = token_embedding.shape
    n = bsz * seq
    t = token_embedding.astype(jnp.bfloat16)
    t = t.reshape(v, d // 32, 2, 16).swapaxes(2, 3).reshape(v, d // 2, 2)
    table_i32 = jax.lax.bitcast_convert_type(t, jnp.int32)
    tok3d = integer_tokens.reshape(_NW, n // (_NW * _CHUNK), _CHUNK)
    out = _sc_embedding_gather(tok3d, table_i32, n, d)
    return out.reshape(bsz, seq, d)


# speedup vs baseline: 1.2143x; 1.2143x over previous
import functools

import jax
import jax.numpy as jnp
from jax import lax
from jax.experimental import pallas as pl
from jax.experimental.pallas import tpu as pltpu
from jax.experimental.pallas import tpu_sc as plsc

_NC = 2
_NS = 16
_NW = _NC * _NS
_CHUNK = 128
_SLOTS = 2


@functools.partial(jax.jit, static_argnums=(2, 3))
def _sc_embedding_gather(tokens_3d, table, b, d):
    b_per_w = b // _NW
    n_chunks = b_per_w // _CHUNK
    mesh = plsc.VectorSubcoreMesh(core_axis_name="c", subcore_axis_name="s")

    @functools.partial(
        pl.kernel,
        mesh=mesh,
        out_type=jax.ShapeDtypeStruct((b, d), jnp.float32),
        scratch_types=(
            [pltpu.VMEM((n_chunks, _CHUNK), jnp.int32)]
            + [pltpu.VMEM((_CHUNK, d), jnp.float32) for _ in range(_SLOTS)]
            + [pltpu.SemaphoreType.DMA for _ in range(2 * _SLOTS)]
        ),
    )
    def k(tok_hbm, tab_hbm, out_hbm, idx_v, *rest):
        rows = rest[:_SLOTS]
        gsem = rest[_SLOTS:2 * _SLOTS]
        wsem = rest[2 * _SLOTS:]
        wid = lax.axis_index("s") * _NC + lax.axis_index("c")
        base = wid * b_per_w

        pltpu.sync_copy(tok_hbm.at[wid], idx_v)

        def gather_start(c, p):
            pltpu.make_async_copy(tab_hbm.at[idx_v.at[c]], rows[p], gsem[p]).start()

        def gather_wait(p):
            pltpu.make_async_copy(tab_hbm.at[idx_v.at[0]], rows[p], gsem[p]).wait()

        def write_start(c, p):
            pltpu.make_async_copy(
                rows[p], out_hbm.at[pl.ds(base + c * _CHUNK, _CHUNK)], wsem[p]
            ).start()

        def write_wait(p):
            pltpu.make_async_copy(
                rows[p], out_hbm.at[pl.ds(base, _CHUNK)], wsem[p]
            ).wait()

        for p in range(_SLOTS):
            gather_start(p, p)

        def body(j, carry):
            c0 = _SLOTS * j
            for p in range(_SLOTS):
                gather_wait(p)
                write_start(c0 + p, p)
            for p in range(_SLOTS):
                write_wait(p)
                gather_start(lax.min(c0 + _SLOTS + p, n_chunks - 1), p)
            return carry

        lax.fori_loop(0, n_chunks // _SLOTS, body, 0)
        for p in range(_SLOTS):
            gather_wait(p)

    return k(tokens_3d, table)


def kernel(integer_tokens, token_embedding):
    bsz, seq = integer_tokens.shape
    d = token_embedding.shape[1]
    n = bsz * seq
    tok3d = integer_tokens.reshape(_NW, n // (_NW * _CHUNK), _CHUNK)
    out = _sc_embedding_gather(tok3d, token_embedding, n, d)
    return out.reshape(bsz, seq, d)
